# baseline (speedup 1.0000x reference)
import functools

import jax
import jax.numpy as jnp
from jax import lax
from jax.experimental import pallas as pl
from jax.experimental.pallas import tpu as pltpu
from jax.experimental.pallas import tpu_sc as plsc

_K = 8192
_D = 32
_N = 8192
_R = 512
_G = _N // _R
_BETA = 0.25

_NC, _NS = 2, 16
_NW = _NC * _NS
_RW = _N // _NW
_IC = 128

_T = 128
_NT = _K // _T


def _tc_body(x_ref, br_ref, er_ref, idx_ref):
    x = x_ref[...]
    x2 = x + x
    a = jnp.sum(x * x, axis=1, keepdims=True)
    minv = None
    jmin = None
    for j in range(_NT):
        et = er_ref[pl.ds(j * _T, _T), :]
        m2t = lax.dot_general(x2, et, (((1,), (1,)), ((), ())),
                              preferred_element_type=jnp.float32)
        d = (a + br_ref[:, pl.ds(j * _T, _T)]) - m2t
        if j == 0:
            minv = d
            jmin = jnp.zeros((_R, _T), jnp.float32)
        else:
            c = d < minv
            minv = jnp.where(c, d, minv)
            jmin = jnp.where(c, jnp.float32(j), jmin)
    gmin = jnp.min(minv, axis=1, keepdims=True)
    lane = lax.broadcasted_iota(jnp.int32, (_R, _T), 1).astype(jnp.float32)
    key = jmin * jnp.float32(_T) + lane
    idx = jnp.min(jnp.where(minv == gmin, key, jnp.float32(_K)), axis=1)
    idx_ref[...] = idx.astype(jnp.int32)


_tc_call = pl.pallas_call(
    _tc_body,
    grid=(_G,),
    in_specs=[
        pl.BlockSpec((_R, _D), lambda i: (i, 0)),
        pl.BlockSpec((1, _K), lambda i: (0, 0)),
        pl.BlockSpec((_K, _D), lambda i: (0, 0)),
    ],
    out_specs=pl.BlockSpec((_R,), lambda i: (i,)),
    out_shape=jax.ShapeDtypeStruct((_N,), jnp.int32),
)


_sc_mesh = plsc.VectorSubcoreMesh(core_axis_name="c", subcore_axis_name="s")


@functools.partial(
    pl.kernel,
    out_type=[
        jax.ShapeDtypeStruct((_N, _D), jnp.float32),
        jax.ShapeDtypeStruct((_NW * 16,), jnp.float32),
    ],
    mesh=_sc_mesh,
    compiler_params=pltpu.CompilerParams(use_tc_tiling_on_sc=False),
    scratch_types=[
        pltpu.VMEM((_IC,), jnp.int32),
        pltpu.VMEM((_IC,), jnp.int32),
        pltpu.VMEM((_RW, _D), jnp.float32),
        pltpu.VMEM((_RW, _D), jnp.float32),
        pltpu.VMEM((16,), jnp.float32),
        pltpu.SemaphoreType.DMA,
    ],
)
def _sc_gather(e_hbm, idx_hbm, x_hbm, out_hbm, ls_hbm,
               idx_v0, idx_v1, q_v, x_v, acc_v, sem):
    wid = lax.axis_index("s") * _NC + lax.axis_index("c")
    base = wid * _RW
    pltpu.sync_copy(idx_hbm.at[pl.ds(base, _IC)], idx_v0)
    pltpu.sync_copy(idx_hbm.at[pl.ds(base + _IC, _IC)], idx_v1)
    pltpu.sync_copy(x_hbm.at[pl.ds(base, _RW), :], x_v)
    cp0 = pltpu.async_copy(e_hbm.at[idx_v0], q_v.at[pl.ds(0, _IC), :], sem)
    cp1 = pltpu.async_copy(e_hbm.at[idx_v1], q_v.at[pl.ds(_IC, _IC), :], sem)
    cp0.wait()
    cp1.wait()

    def body(i, acc):
        xa = x_v[i, pl.ds(0, 16)]
        da = q_v[i, pl.ds(0, 16)] - xa
        x_v[i, pl.ds(0, 16)] = xa + da
        xb = x_v[i, pl.ds(16, 16)]
        db = q_v[i, pl.ds(16, 16)] - xb
        x_v[i, pl.ds(16, 16)] = xb + db
        return acc + da * da + db * db

    acc = lax.fori_loop(0, _RW, body, jnp.zeros((16,), jnp.float32))
    acc_v[...] = acc
    pltpu.sync_copy(x_v, out_hbm.at[pl.ds(base, _RW), :])
    pltpu.sync_copy(acc_v, ls_hbm.at[pl.ds(wid * 16, 16)])


def kernel(inputs, embedding_weight):
    x = jnp.transpose(inputs, (0, 2, 3, 1)).reshape(_N, _D)
    b = jnp.sum(embedding_weight ** 2, axis=1).reshape(1, _K)
    idx = _tc_call(x, b, embedding_weight)
    out2d, ls = _sc_gather(embedding_weight, idx, x)
    loss = (1.0 + _BETA) * jnp.sum(ls) / jnp.float32(_N * _D)
    out = out2d.reshape(8, 32, 32, _D).transpose(0, 3, 1, 2)
    return (loss, out)


# speedup vs baseline: 1.0224x; 1.0224x over previous
import functools

import jax
import jax.numpy as jnp
from jax import lax
from jax.experimental import pallas as pl
from jax.experimental.pallas import tpu as pltpu
from jax.experimental.pallas import tpu_sc as plsc

_K = 8192
_D = 32
_N = 8192
_R = 256
_G = _N // _R
_BETA = 0.25

_NC, _NS = 2, 16
_NW = _NC * _NS
_RW = _N // _NW
_IC = 128

_T = 128
_NT = _K // _T


def _tc_body(x_ref, br_ref, er_ref, idx_ref):
    x = x_ref[...]
    x2 = x + x
    a = jnp.sum(x * x, axis=1, keepdims=True)
    minv = None
    jmin = None
    for j in range(_NT):
        et = er_ref[pl.ds(j * _T, _T), :]
        m2t = lax.dot_general(x2, et, (((1,), (1,)), ((), ())),
                              preferred_element_type=jnp.float32)
        d = (a + br_ref[:, pl.ds(j * _T, _T)]) - m2t
        if j == 0:
            minv = d
            jmin = jnp.zeros((_R, _T), jnp.float32)
        else:
            c = d < minv
            minv = jnp.where(c, d, minv)
            jmin = jnp.where(c, jnp.float32(j), jmin)
    gmin = jnp.min(minv, axis=1, keepdims=True)
    lane = lax.broadcasted_iota(jnp.int32, (_R, _T), 1).astype(jnp.float32)
    key = jmin * jnp.float32(_T) + lane
    idx = jnp.min(jnp.where(minv == gmin, key, jnp.float32(_K)), axis=1)
    idx_ref[...] = idx.astype(jnp.int32)


_tc_call = pl.pallas_call(
    _tc_body,
    grid=(_G,),
    in_specs=[
        pl.BlockSpec((_R, _D), lambda i: (i, 0)),
        pl.BlockSpec((1, _K), lambda i: (0, 0)),
        pl.BlockSpec((_K, _D), lambda i: (0, 0)),
    ],
    out_specs=pl.BlockSpec((_R,), lambda i: (i,)),
    out_shape=jax.ShapeDtypeStruct((_N,), jnp.int32),
)


_sc_mesh = plsc.VectorSubcoreMesh(core_axis_name="c", subcore_axis_name="s")


@functools.partial(
    pl.kernel,
    out_type=[
        jax.ShapeDtypeStruct((_N, _D), jnp.float32),
        jax.ShapeDtypeStruct((_NW * 16,), jnp.float32),
    ],
    mesh=_sc_mesh,
    compiler_params=pltpu.CompilerParams(use_tc_tiling_on_sc=False),
    scratch_types=[
        pltpu.VMEM((_IC,), jnp.int32),
        pltpu.VMEM((_IC,), jnp.int32),
        pltpu.VMEM((_RW, _D), jnp.float32),
        pltpu.VMEM((_RW, _D), jnp.float32),
        pltpu.VMEM((16,), jnp.float32),
        pltpu.SemaphoreType.DMA,
    ],
)
def _sc_gather(e_hbm, idx_hbm, x_hbm, out_hbm, ls_hbm,
               idx_v0, idx_v1, q_v, x_v, acc_v, sem):
    wid = lax.axis_index("s") * _NC + lax.axis_index("c")
    base = wid * _RW
    pltpu.sync_copy(idx_hbm.at[pl.ds(base, _IC)], idx_v0)
    pltpu.sync_copy(idx_hbm.at[pl.ds(base + _IC, _IC)], idx_v1)
    pltpu.sync_copy(x_hbm.at[pl.ds(base, _RW), :], x_v)
    cp0 = pltpu.async_copy(e_hbm.at[idx_v0], q_v.at[pl.ds(0, _IC), :], sem)
    cp1 = pltpu.async_copy(e_hbm.at[idx_v1], q_v.at[pl.ds(_IC, _IC), :], sem)
    cp0.wait()
    cp1.wait()

    def body(i, acc):
        xa = x_v[i, pl.ds(0, 16)]
        da = q_v[i, pl.ds(0, 16)] - xa
        x_v[i, pl.ds(0, 16)] = xa + da
        xb = x_v[i, pl.ds(16, 16)]
        db = q_v[i, pl.ds(16, 16)] - xb
        x_v[i, pl.ds(16, 16)] = xb + db
        return acc + da * da + db * db

    acc = lax.fori_loop(0, _RW, body, jnp.zeros((16,), jnp.float32))
    acc_v[...] = acc
    pltpu.sync_copy(x_v, out_hbm.at[pl.ds(base, _RW), :])
    pltpu.sync_copy(acc_v, ls_hbm.at[pl.ds(wid * 16, 16)])


def kernel(inputs, embedding_weight):
    x = jnp.transpose(inputs, (0, 2, 3, 1)).reshape(_N, _D)
    b = jnp.sum(embedding_weight ** 2, axis=1).reshape(1, _K)
    idx = _tc_call(x, b, embedding_weight)
    out2d, ls = _sc_gather(embedding_weight, idx, x)
    loss = (1.0 + _BETA) * jnp.sum(ls) / jnp.float32(_N * _D)
    out = out2d.reshape(8, 32, 32, _D).transpose(0, 3, 1, 2)
    return (loss, out)
